# baseline (speedup 1.0000x reference)
import jax
import jax.numpy as jnp
import numpy as np
from jax import lax
from jax.experimental import pallas as pl
from jax.experimental.pallas import tpu as pltpu
from jax.experimental.pallas import tpu_sc as plsc

_FACTOR = 8

_rng = np.random.RandomState(0)
_FIXED_IDX = np.stack(
    [_rng.choice(_FACTOR, _FACTOR, replace=False),
     _rng.choice(_FACTOR, _FACTOR, replace=False)],
).astype(np.int32)

_R_BLOCK = 10896


def _body(img_ref, out_img_ref):
    out_img_ref[...] = img_ref[...]


def _idx_sc_body(idx_hbm, out_idx_hbm, idx_s, out_s):
    c = lax.axis_index("c")

    @pl.when(c == 0)
    def _():
        pltpu.sync_copy(idx_hbm, idx_s)
        nz = idx_s[0] != 0
        for k in range(1, 16):
            nz = nz | (idx_s[k] != 0)
        for k in range(16):
            out_s[k] = jnp.where(nz, idx_s[k],
                                 jnp.int32(_FIXED_IDX.ravel()[k]))
        pltpu.sync_copy(out_s, out_idx_hbm)


def kernel(img, indices):
    c, h, w = img.shape
    img2 = img.reshape(c * h, w)

    out2 = pl.pallas_call(
        _body,
        grid=(pl.cdiv(c * h, _R_BLOCK),),
        in_specs=[pl.BlockSpec((_R_BLOCK, w), lambda i: (i, 0))],
        out_specs=pl.BlockSpec((_R_BLOCK, w), lambda i: (i, 0)),
        compiler_params=pltpu.CompilerParams(
            vmem_limit_bytes=64 * 1024 * 1024),
        out_shape=jax.ShapeDtypeStruct((c * h, w), img.dtype),
    )(img2)

    idx_sc = pl.kernel(
        _idx_sc_body,
        out_type=jax.ShapeDtypeStruct((16,), jnp.int32),
        mesh=plsc.ScalarSubcoreMesh(axis_name="c", num_cores=2),
        scratch_types=[
            pltpu.SMEM((16,), jnp.int32),
            pltpu.SMEM((16,), jnp.int32),
        ],
    )
    out_idx = idx_sc(indices.reshape(16)).reshape(2, _FACTOR)
    return out2.reshape(c, h, w), out_idx


# speedup vs baseline: 1.5105x; 1.5105x over previous
import jax
import jax.numpy as jnp
import numpy as np
from jax.experimental import pallas as pl
from jax.experimental.pallas import tpu as pltpu

_FACTOR = 8

_rng = np.random.RandomState(0)
_FIXED_IDX = np.stack(
    [_rng.choice(_FACTOR, _FACTOR, replace=False),
     _rng.choice(_FACTOR, _FACTOR, replace=False)],
).astype(np.int32)

_R_BLOCK = 10896


def _body(idx_ref, img_ref, out_img_ref, out_idx_ref):
    out_img_ref[...] = img_ref[...]

    @pl.when(pl.program_id(0) == 0)
    def _():
        nz = idx_ref[0, 0] != 0
        for i in range(2):
            for j in range(_FACTOR):
                if (i, j) != (0, 0):
                    nz = nz | (idx_ref[i, j] != 0)
        for i in range(2):
            for j in range(_FACTOR):
                out_idx_ref[i, j] = jnp.where(
                    nz, idx_ref[i, j], jnp.int32(_FIXED_IDX[i, j]))


def kernel(img, indices):
    c, h, w = img.shape
    img2 = img.reshape(c * h, w)

    out2, out_idx = pl.pallas_call(
        _body,
        grid=(pl.cdiv(c * h, _R_BLOCK),),
        in_specs=[
            pl.BlockSpec(memory_space=pltpu.SMEM),
            pl.BlockSpec((_R_BLOCK, w), lambda i: (i, 0)),
        ],
        out_specs=[
            pl.BlockSpec((_R_BLOCK, w), lambda i: (i, 0)),
            pl.BlockSpec(memory_space=pltpu.SMEM),
        ],
        compiler_params=pltpu.CompilerParams(
            vmem_limit_bytes=64 * 1024 * 1024),
        out_shape=[
            jax.ShapeDtypeStruct((c * h, w), img.dtype),
            jax.ShapeDtypeStruct((2, _FACTOR), jnp.int32),
        ],
    )(indices, img2)
    return out2.reshape(c, h, w), out_idx
